# Initial kernel scaffold; baseline (speedup 1.0000x reference)
import jax
import jax.numpy as jnp
from jax.experimental import pallas as pl


def kernel(x, edge_index, W1, b1, W2, b2, cW1, cb1, cW2, cb2, gW1, gb1, gW2, gb2):
    raise NotImplementedError("write your pallas kernel here")


# speedup vs baseline: 34.4419x; 34.4419x over previous
import functools

import jax
import jax.numpy as jnp
from jax import lax
from jax.experimental import pallas as pl
from jax.experimental.pallas import tpu as pltpu
from jax.experimental.pallas import tpu_sc as plsc

N = 100000
E = 3200000
FULL_DIM = 128
GRAPH_DIM = 3
HID = 16
OUT = 64

NC = 2
NS = 16
NW = NC * NS

NPAD = 100352
EPAD = NW * NPAD
TPW = EPAD // NW
RPW = TPW // 128
KROWS = 8
ITERS = RPW // KROWS
ZROWS = NPAD // NS
SENT = NPAD - 1

_MESH = plsc.VectorSubcoreMesh(core_axis_name="c", subcore_axis_name="s")
_HIGH = jax.lax.Precision.HIGHEST
_SC_PARAMS = pltpu.CompilerParams(use_tc_tiling_on_sc=False)


def _fill(ref, nvec, value):
    v = jnp.full((16,), value, jnp.float32)

    def body(i, _):
        ref[pl.ds(i * 16, 16)] = v
        return 0

    lax.fori_loop(0, nvec, body, 0)


def _deg_body(dst2, zrow, degp, didx, ones_v, acc):
    c = lax.axis_index("c")
    s = lax.axis_index("s")
    wid = c * NS + s

    _fill(ones_v, 8, 1.0)
    pltpu.sync_copy(zrow, acc.at[pl.ds(s * ZROWS, ZROWS)])
    plsc.subcore_barrier()

    def body(it, _):
        row0 = wid * RPW + it * KROWS
        pltpu.sync_copy(dst2.at[pl.ds(row0, KROWS)], didx)
        for j in range(KROWS):
            pltpu.sync_copy(ones_v, acc.at[didx.at[j]], add=True)
        return 0

    lax.fori_loop(0, ITERS, body, 0)
    plsc.subcore_barrier()
    pltpu.sync_copy(acc.at[pl.ds(s * ZROWS, ZROWS)],
                    degp.at[c, pl.ds(s * ZROWS, ZROWS)])


_deg_kernel = pl.kernel(
    _deg_body,
    out_type=jax.ShapeDtypeStruct((NC, NPAD), jnp.float32),
    mesh=_MESH,
    compiler_params=_SC_PARAMS,
    scratch_types=[
        pltpu.VMEM((KROWS, 128), jnp.int32),
        pltpu.VMEM((128,), jnp.float32),
        pltpu.VMEM_SHARED((NPAD,), jnp.float32),
    ],
)


def _make_agg(W, stage_table):
    def body(table, src2, dst2, zrows, aggp, sidx, didx, msgs, acc, *rest):
        if stage_table:
            table_sh, sem = rest
        else:
            (sem,) = rest
        c = lax.axis_index("c")
        s = lax.axis_index("s")
        wid = c * NS + s

        pltpu.sync_copy(zrows, acc.at[pl.ds(s * ZROWS, ZROWS), :])
        if stage_table:
            pltpu.sync_copy(table.at[pl.ds(s * ZROWS, ZROWS), :],
                            table_sh.at[pl.ds(s * ZROWS, ZROWS), :])
        plsc.subcore_barrier()

        gsrc = table_sh if stage_table else table

        def body_it(it, _):
            row0 = wid * RPW + it * KROWS
            pltpu.sync_copy(src2.at[pl.ds(row0, KROWS)], sidx)
            pltpu.sync_copy(dst2.at[pl.ds(row0, KROWS)], didx)
            descs = [
                pltpu.async_copy(gsrc.at[sidx.at[j]], msgs.at[j], sem)
                for j in range(KROWS)
            ]
            for d in descs:
                d.wait()
            for j in range(KROWS):
                pltpu.sync_copy(msgs.at[j], acc.at[didx.at[j]], add=True)
            return 0

        lax.fori_loop(0, ITERS, body_it, 0)
        plsc.subcore_barrier()
        pltpu.sync_copy(acc.at[pl.ds(s * ZROWS, ZROWS), :],
                        aggp.at[c, pl.ds(s * ZROWS, ZROWS), :])

    scratch = [
        pltpu.VMEM((KROWS, 128), jnp.int32),
        pltpu.VMEM((KROWS, 128), jnp.int32),
        pltpu.VMEM((KROWS, 128, W), jnp.float32),
        pltpu.VMEM_SHARED((NPAD, W), jnp.float32),
    ]
    if stage_table:
        scratch.append(pltpu.VMEM_SHARED((NPAD, W), jnp.float32))
    scratch.append(pltpu.SemaphoreType.DMA)

    return pl.kernel(
        body,
        out_type=jax.ShapeDtypeStruct((NC, NPAD, W), jnp.float32),
        mesh=_MESH,
        compiler_params=_SC_PARAMS,
        scratch_types=scratch,
    )


_agg16 = _make_agg(16, stage_table=False)


_R = 1000


def _tc1_body(x, d0, d1, P16, cW1p, cb1, cW2, cb2, gW1, gb1, gv, gb,
              dinv_o, xgs_o, w0_o, pre_o):
    deg = d0[...] + d1[...] + 1.0
    dinv = lax.rsqrt(deg)
    xv = x[...]
    dinv_o[...] = dinv
    xgs_o[...] = jnp.dot(xv, P16[...], precision=_HIGH) * dinv
    ch = jnp.maximum(jnp.dot(xv, cW1p[...], precision=_HIGH) + cb1[...], 0.0)
    ctx = jnp.dot(ch, cW2[...], precision=_HIGH) + cb2[...]
    gh = jnp.maximum(jnp.dot(xv, gW1[...], precision=_HIGH) + gb1[...], 0.0)
    dl = jnp.dot(gh, gv[...], precision=_HIGH) + gb[...]
    w0 = 1.0 / (1.0 + jnp.exp(-dl))
    w0_o[...] = w0
    pre_o[...] = (1.0 - w0) * ctx


def _bs(shape, idx):
    return pl.BlockSpec(shape, idx)


_row = lambda i: (i, 0)
_rep = lambda i: (0, 0)

_tc1 = pl.pallas_call(
    _tc1_body,
    grid=(N // _R,),
    in_specs=[
        _bs((_R, FULL_DIM), _row),
        _bs((_R, 1), _row), _bs((_R, 1), _row),
        _bs((FULL_DIM, HID), _rep),
        _bs((FULL_DIM, HID), _rep), _bs((1, HID), _rep),
        _bs((HID, OUT), _rep), _bs((1, OUT), _rep),
        _bs((FULL_DIM, HID), _rep), _bs((1, HID), _rep),
        _bs((HID, 1), _rep), _bs((1, 1), _rep),
    ],
    out_specs=[
        _bs((_R, 1), _row), _bs((_R, HID), _row),
        _bs((_R, 1), _row), _bs((_R, OUT), _row),
    ],
    out_shape=[
        jax.ShapeDtypeStruct((N, 1), jnp.float32),
        jax.ShapeDtypeStruct((N, HID), jnp.float32),
        jax.ShapeDtypeStruct((N, 1), jnp.float32),
        jax.ShapeDtypeStruct((N, OUT), jnp.float32),
    ],
)


def _tc2_body(g1a, g1b, xgs, dinv, W1p, b1, hs_o):
    dv = dinv[...]
    s1 = (g1a[...] + g1b[...] + xgs[...]) * dv
    h = jnp.maximum(jnp.dot(s1, W1p[...], precision=_HIGH) + b1[...], 0.0)
    hs_o[...] = h * dv


_tc2 = pl.pallas_call(
    _tc2_body,
    grid=(N // _R,),
    in_specs=[
        _bs((_R, HID), _row), _bs((_R, HID), _row), _bs((_R, HID), _row),
        _bs((_R, 1), _row),
        _bs((HID, HID), _rep), _bs((1, HID), _rep),
    ],
    out_specs=[_bs((_R, HID), _row)],
    out_shape=[jax.ShapeDtypeStruct((N, HID), jnp.float32)],
)


def _tc3_body(g2a, g2b, hs, dinv, w0, pre, W2, b2, out_o):
    s2 = (g2a[...] + g2b[...] + hs[...]) * dinv[...]
    go = jnp.dot(s2, W2[...], precision=_HIGH) + b2[...]
    out_o[...] = w0[...] * go + pre[...]


_tc3 = pl.pallas_call(
    _tc3_body,
    grid=(N // _R,),
    in_specs=[
        _bs((_R, HID), _row), _bs((_R, HID), _row), _bs((_R, HID), _row),
        _bs((_R, 1), _row), _bs((_R, 1), _row), _bs((_R, OUT), _row),
        _bs((HID, OUT), _rep), _bs((1, OUT), _rep),
    ],
    out_specs=[_bs((_R, OUT), _row)],
    out_shape=[jax.ShapeDtypeStruct((N, OUT), jnp.float32)],
)


def kernel(x, edge_index, W1, b1, W2, b2, cW1, cb1, cW2, cb2, gW1, gb1,
           gW2, gb2):
    pad = jnp.full((EPAD - E,), SENT, jnp.int32)
    src2 = jnp.concatenate([edge_index[0], pad]).reshape(EPAD // 128, 128)
    dst2 = jnp.concatenate([edge_index[1], pad]).reshape(EPAD // 128, 128)

    P16 = jnp.zeros((FULL_DIM, HID), jnp.float32).at[:GRAPH_DIM, :GRAPH_DIM].set(
        jnp.eye(GRAPH_DIM, dtype=jnp.float32))
    cW1p = jnp.zeros((FULL_DIM, HID), jnp.float32).at[GRAPH_DIM:, :].set(cW1)
    W1p = jnp.zeros((HID, HID), jnp.float32).at[:GRAPH_DIM, :].set(W1)
    gv = (gW2[:, 0] - gW2[:, 1]).reshape(HID, 1)
    gb = (gb2[0] - gb2[1]).reshape(1, 1)

    zrow1 = jnp.zeros((ZROWS,), jnp.float32)
    zrow16 = jnp.zeros((ZROWS, HID), jnp.float32)
    degp = _deg_kernel(dst2, zrow1)
    d0 = degp[0, :N].reshape(N, 1)
    d1 = degp[1, :N].reshape(N, 1)

    dinv, xgs, w0, pre = _tc1(x, d0, d1, P16, cW1p, cb1.reshape(1, HID),
                              cW2, cb2.reshape(1, OUT), gW1,
                              gb1.reshape(1, HID), gv, gb)

    xgs_pad = jnp.pad(xgs, ((0, NPAD - N), (0, 0)))
    g1 = _agg16(xgs_pad, src2, dst2, zrow16)

    hs = _tc2(g1[0, :N], g1[1, :N], xgs, dinv, W1p, b1.reshape(1, HID))[0]

    hs_pad = jnp.pad(hs, ((0, NPAD - N), (0, 0)))
    g2 = _agg16(hs_pad, src2, dst2, zrow16)

    out = _tc3(g2[0, :N], g2[1, :N], hs, dinv, w0, pre, W2,
               b2.reshape(1, OUT))[0]
    return out


# speedup vs baseline: 39.0136x; 1.1327x over previous
import functools

import jax
import jax.numpy as jnp
from jax import lax
from jax.experimental import pallas as pl
from jax.experimental.pallas import tpu as pltpu
from jax.experimental.pallas import tpu_sc as plsc

N = 100000
E = 3200000
FULL_DIM = 128
GRAPH_DIM = 3
HID = 16
OUT = 64

NC = 2
NS = 16
NW = NC * NS

NPAD = 100352
EPAD = NW * NPAD
TPW = EPAD // NW
RPW = TPW // 128
KROWS = 8
ITERS = RPW // KROWS
ZROWS = NPAD // NS
SENT = NPAD - 1

_MESH = plsc.VectorSubcoreMesh(core_axis_name="c", subcore_axis_name="s")
_HIGH = jax.lax.Precision.HIGHEST
_SC_PARAMS = pltpu.CompilerParams(use_tc_tiling_on_sc=False)


def _fill(ref, nvec, value):
    v = jnp.full((16,), value, jnp.float32)

    def body(i, _):
        ref[pl.ds(i * 16, 16)] = v
        return 0

    lax.fori_loop(0, nvec, body, 0)


def _deg_body(dst2, zrow, degp, didx, ones_v, acc):
    c = lax.axis_index("c")
    s = lax.axis_index("s")
    wid = c * NS + s

    _fill(ones_v, 8, 1.0)
    pltpu.sync_copy(zrow, acc.at[pl.ds(s * ZROWS, ZROWS)])
    plsc.subcore_barrier()

    def body(it, _):
        row0 = wid * RPW + it * KROWS
        pltpu.sync_copy(dst2.at[pl.ds(row0, KROWS)], didx)
        for j in range(KROWS):
            pltpu.sync_copy(ones_v, acc.at[didx.at[j]], add=True)
        return 0

    lax.fori_loop(0, ITERS, body, 0)
    plsc.subcore_barrier()
    pltpu.sync_copy(acc.at[pl.ds(s * ZROWS, ZROWS)],
                    degp.at[c, pl.ds(s * ZROWS, ZROWS)])


_deg_kernel = pl.kernel(
    _deg_body,
    out_type=jax.ShapeDtypeStruct((NC, NPAD), jnp.float32),
    mesh=_MESH,
    compiler_params=_SC_PARAMS,
    scratch_types=[
        pltpu.VMEM((KROWS, 128), jnp.int32),
        pltpu.VMEM((128,), jnp.float32),
        pltpu.VMEM_SHARED((NPAD,), jnp.float32),
    ],
)


KRA = 4
PHASES = RPW // KRA


def _make_agg(W):
    def body(table, src2, dst2, zrows, aggp, acc,
             si0, si1, si2, di0, di1, di2, ms0, ms1,
             isem, gsem, ssem):
        sidx = [si0, si1, si2]
        didx = [di0, di1, di2]
        msgs = [ms0, ms1]
        c = lax.axis_index("c")
        s = lax.axis_index("s")
        wid = c * NS + s
        base = wid * RPW

        pltpu.sync_copy(zrows, acc.at[pl.ds(s * ZROWS, ZROWS), :])
        plsc.subcore_barrier()

        def iload(t, ib):
            row0 = base + t * KRA
            pltpu.async_copy(src2.at[pl.ds(row0, KRA)], sidx[ib], isem)
            pltpu.async_copy(dst2.at[pl.ds(row0, KRA)], didx[ib], isem)

        def iwait(ib):
            pltpu.make_async_copy(src2.at[pl.ds(0, KRA)], sidx[ib],
                                  isem).wait()
            pltpu.make_async_copy(dst2.at[pl.ds(0, KRA)], didx[ib],
                                  isem).wait()

        def gathers(ib, mb):
            for j in range(KRA):
                pltpu.async_copy(table.at[sidx[ib].at[j]], msgs[mb].at[j],
                                 gsem)

        def gwait(mb):
            for j in range(KRA):
                pltpu.make_async_copy(table.at[sidx[0].at[j]],
                                      msgs[mb].at[j], gsem).wait()

        def scat(ib, mb):
            for j in range(KRA):
                pltpu.async_copy(msgs[mb].at[j], acc.at[didx[ib].at[j]],
                                 ssem, add=True)

        def swait():
            for j in range(KRA):
                pltpu.make_async_copy(msgs[0].at[j],
                                      acc.at[didx[0].at[j]], ssem).wait()

        iload(0, 0)
        iwait(0)
        gathers(0, 0)
        iload(1, 1)
        iwait(1)
        iload(2, 2)
        gwait(0)
        gathers(1, 1)
        scat(0, 0)
        swait()
        iwait(2)
        iload(3, 0)
        gwait(1)
        gathers(2, 0)
        scat(1, 1)

        def body_m(m, _):
            t0 = 2 + 6 * m
            for i in range(6):
                mb = i % 2
                ib = (i + 2) % 3
                swait()
                iwait(i % 3)
                iload(t0 + i + 2, (i + 1) % 3)
                gwait(mb)
                gathers(i % 3, (i + 1) % 2)
                scat(ib, mb)
            return 0

        lax.fori_loop(0, (PHASES - 4) // 6, body_m, 0)

        swait()
        iwait(0)
        gwait(0)
        gathers(0, 1)
        scat(2, 0)
        swait()
        gwait(1)
        scat(0, 1)
        swait()

        plsc.subcore_barrier()
        pltpu.sync_copy(acc.at[pl.ds(s * ZROWS, ZROWS), :],
                        aggp.at[c, pl.ds(s * ZROWS, ZROWS), :])

    scratch = [
        pltpu.VMEM_SHARED((NPAD, W), jnp.float32),
        pltpu.VMEM((KRA, 128), jnp.int32),
        pltpu.VMEM((KRA, 128), jnp.int32),
        pltpu.VMEM((KRA, 128), jnp.int32),
        pltpu.VMEM((KRA, 128), jnp.int32),
        pltpu.VMEM((KRA, 128), jnp.int32),
        pltpu.VMEM((KRA, 128), jnp.int32),
        pltpu.VMEM((KRA, 128, W), jnp.float32),
        pltpu.VMEM((KRA, 128, W), jnp.float32),
        pltpu.SemaphoreType.DMA,
        pltpu.SemaphoreType.DMA,
        pltpu.SemaphoreType.DMA,
    ]

    return pl.kernel(
        body,
        out_type=jax.ShapeDtypeStruct((NC, NPAD, W), jnp.float32),
        mesh=_MESH,
        compiler_params=_SC_PARAMS,
        scratch_types=scratch,
    )


_agg16 = _make_agg(16)


_R = 1000


def _tc1_body(x, d0, d1, P16, cW1p, cb1, cW2, cb2, gW1, gb1, gv, gb,
              dinv_o, xgs_o, w0_o, pre_o):
    deg = d0[...] + d1[...] + 1.0
    dinv = lax.rsqrt(deg)
    xv = x[...]
    dinv_o[...] = dinv
    xgs_o[...] = jnp.dot(xv, P16[...], precision=_HIGH) * dinv
    ch = jnp.maximum(jnp.dot(xv, cW1p[...], precision=_HIGH) + cb1[...], 0.0)
    ctx = jnp.dot(ch, cW2[...], precision=_HIGH) + cb2[...]
    gh = jnp.maximum(jnp.dot(xv, gW1[...], precision=_HIGH) + gb1[...], 0.0)
    dl = jnp.dot(gh, gv[...], precision=_HIGH) + gb[...]
    w0 = 1.0 / (1.0 + jnp.exp(-dl))
    w0_o[...] = w0
    pre_o[...] = (1.0 - w0) * ctx


def _bs(shape, idx):
    return pl.BlockSpec(shape, idx)


_row = lambda i: (i, 0)
_rep = lambda i: (0, 0)

_tc1 = pl.pallas_call(
    _tc1_body,
    grid=(N // _R,),
    in_specs=[
        _bs((_R, FULL_DIM), _row),
        _bs((_R, 1), _row), _bs((_R, 1), _row),
        _bs((FULL_DIM, HID), _rep),
        _bs((FULL_DIM, HID), _rep), _bs((1, HID), _rep),
        _bs((HID, OUT), _rep), _bs((1, OUT), _rep),
        _bs((FULL_DIM, HID), _rep), _bs((1, HID), _rep),
        _bs((HID, 1), _rep), _bs((1, 1), _rep),
    ],
    out_specs=[
        _bs((_R, 1), _row), _bs((_R, HID), _row),
        _bs((_R, 1), _row), _bs((_R, OUT), _row),
    ],
    out_shape=[
        jax.ShapeDtypeStruct((N, 1), jnp.float32),
        jax.ShapeDtypeStruct((N, HID), jnp.float32),
        jax.ShapeDtypeStruct((N, 1), jnp.float32),
        jax.ShapeDtypeStruct((N, OUT), jnp.float32),
    ],
)


def _tc2_body(g1a, g1b, xgs, dinv, W1p, b1, hs_o):
    dv = dinv[...]
    s1 = (g1a[...] + g1b[...] + xgs[...]) * dv
    h = jnp.maximum(jnp.dot(s1, W1p[...], precision=_HIGH) + b1[...], 0.0)
    hs_o[...] = h * dv


_tc2 = pl.pallas_call(
    _tc2_body,
    grid=(N // _R,),
    in_specs=[
        _bs((_R, HID), _row), _bs((_R, HID), _row), _bs((_R, HID), _row),
        _bs((_R, 1), _row),
        _bs((HID, HID), _rep), _bs((1, HID), _rep),
    ],
    out_specs=[_bs((_R, HID), _row)],
    out_shape=[jax.ShapeDtypeStruct((N, HID), jnp.float32)],
)


def _tc3_body(g2a, g2b, hs, dinv, w0, pre, W2, b2, out_o):
    s2 = (g2a[...] + g2b[...] + hs[...]) * dinv[...]
    go = jnp.dot(s2, W2[...], precision=_HIGH) + b2[...]
    out_o[...] = w0[...] * go + pre[...]


_tc3 = pl.pallas_call(
    _tc3_body,
    grid=(N // _R,),
    in_specs=[
        _bs((_R, HID), _row), _bs((_R, HID), _row), _bs((_R, HID), _row),
        _bs((_R, 1), _row), _bs((_R, 1), _row), _bs((_R, OUT), _row),
        _bs((HID, OUT), _rep), _bs((1, OUT), _rep),
    ],
    out_specs=[_bs((_R, OUT), _row)],
    out_shape=[jax.ShapeDtypeStruct((N, OUT), jnp.float32)],
)


def kernel(x, edge_index, W1, b1, W2, b2, cW1, cb1, cW2, cb2, gW1, gb1,
           gW2, gb2):
    pad = jnp.full((EPAD - E,), SENT, jnp.int32)
    src2 = jnp.concatenate([edge_index[0], pad]).reshape(EPAD // 128, 128)
    dst2 = jnp.concatenate([edge_index[1], pad]).reshape(EPAD // 128, 128)

    P16 = jnp.zeros((FULL_DIM, HID), jnp.float32).at[:GRAPH_DIM, :GRAPH_DIM].set(
        jnp.eye(GRAPH_DIM, dtype=jnp.float32))
    cW1p = jnp.zeros((FULL_DIM, HID), jnp.float32).at[GRAPH_DIM:, :].set(cW1)
    W1p = jnp.zeros((HID, HID), jnp.float32).at[:GRAPH_DIM, :].set(W1)
    gv = (gW2[:, 0] - gW2[:, 1]).reshape(HID, 1)
    gb = (gb2[0] - gb2[1]).reshape(1, 1)

    zrow1 = jnp.zeros((ZROWS,), jnp.float32)
    zrow16 = jnp.zeros((ZROWS, HID), jnp.float32)
    degp = _deg_kernel(dst2, zrow1)
    d0 = degp[0, :N].reshape(N, 1)
    d1 = degp[1, :N].reshape(N, 1)

    dinv, xgs, w0, pre = _tc1(x, d0, d1, P16, cW1p, cb1.reshape(1, HID),
                              cW2, cb2.reshape(1, OUT), gW1,
                              gb1.reshape(1, HID), gv, gb)

    xgs_pad = jnp.pad(xgs, ((0, NPAD - N), (0, 0)))
    g1 = _agg16(xgs_pad, src2, dst2, zrow16)

    hs = _tc2(g1[0, :N], g1[1, :N], xgs, dinv, W1p, b1.reshape(1, HID))[0]

    hs_pad = jnp.pad(hs, ((0, NPAD - N), (0, 0)))
    g2 = _agg16(hs_pad, src2, dst2, zrow16)

    out = _tc3(g2[0, :N], g2[1, :N], hs, dinv, w0, pre, W2,
               b2.reshape(1, OUT))[0]
    return out


# speedup vs baseline: 64.3961x; 1.6506x over previous
import jax
import jax.numpy as jnp
from jax import lax
from jax.experimental import pallas as pl
from jax.experimental.pallas import tpu as pltpu
from jax.experimental.pallas import tpu_sc as plsc

N = 100000
E = 3200000
FULL_DIM = 128
GRAPH_DIM = 3
HID = 16
OUT = 64

NC = 2
NS = 16
NW = NC * NS

NPAD = 100352
EPAD = NW * NPAD
TPW = EPAD // NW
RPW = TPW // 128
ZROWS = NPAD // NS
SENT = NPAD - 1

BLK = 1024
GRID = NPAD // BLK
PACK = BLK * HID // 128
LIN = NPAD * HID // 128
LROWS = NPAD // 128

_MESH = plsc.VectorSubcoreMesh(core_axis_name="c", subcore_axis_name="s")
_HIGH = jax.lax.Precision.HIGHEST
_SC_PARAMS = pltpu.CompilerParams(use_tc_tiling_on_sc=False)


def _fill(ref, nvec, value):
    v = jnp.full((16,), value, jnp.float32)

    def body(i, _):
        ref[pl.ds(i * 16, 16)] = v
        return 0

    lax.fori_loop(0, nvec, body, 0)


KROWS = 8
ITERS = RPW // KROWS


def _deg_body(dst2, zrow, degp, didx, ones_v, acc):
    c = lax.axis_index("c")
    s = lax.axis_index("s")
    wid = c * NS + s

    _fill(ones_v, 8, 1.0)
    pltpu.sync_copy(zrow, acc.at[pl.ds(s * ZROWS, ZROWS)])
    plsc.subcore_barrier()

    def body(it, _):
        row0 = wid * RPW + it * KROWS
        pltpu.sync_copy(dst2.at[pl.ds(row0, KROWS)], didx)
        for j in range(KROWS):
            pltpu.sync_copy(ones_v, acc.at[didx.at[j]], add=True)
        return 0

    lax.fori_loop(0, ITERS, body, 0)
    plsc.subcore_barrier()
    pltpu.sync_copy(acc.at[pl.ds(s * ZROWS, ZROWS)],
                    degp.at[c, pl.ds(s * ZROWS, ZROWS)])


_deg_kernel = pl.kernel(
    _deg_body,
    out_type=jax.ShapeDtypeStruct((NC, NPAD), jnp.float32),
    mesh=_MESH,
    compiler_params=_SC_PARAMS,
    scratch_types=[
        pltpu.VMEM((KROWS, 128), jnp.int32),
        pltpu.VMEM((128,), jnp.float32),
        pltpu.VMEM_SHARED((NPAD,), jnp.float32),
    ],
)


KRA = 4
PHASES = RPW // KRA


def _make_agg(W):
    def body(table, src2, dst2, zrows, aggp, acc,
             si0, si1, si2, di0, di1, di2, ms0, ms1,
             isem, gsem, ssem):
        sidx = [si0, si1, si2]
        didx = [di0, di1, di2]
        msgs = [ms0, ms1]
        c = lax.axis_index("c")
        s = lax.axis_index("s")
        wid = c * NS + s
        base = wid * RPW

        pltpu.sync_copy(zrows, acc.at[pl.ds(s * ZROWS, ZROWS), :])
        plsc.subcore_barrier()

        def iload(t, ib):
            row0 = base + t * KRA
            pltpu.async_copy(src2.at[pl.ds(row0, KRA)], sidx[ib], isem)
            pltpu.async_copy(dst2.at[pl.ds(row0, KRA)], didx[ib], isem)

        def iwait(ib):
            pltpu.make_async_copy(src2.at[pl.ds(0, KRA)], sidx[ib],
                                  isem).wait()
            pltpu.make_async_copy(dst2.at[pl.ds(0, KRA)], didx[ib],
                                  isem).wait()

        def gathers(ib, mb):
            for j in range(KRA):
                pltpu.async_copy(table.at[sidx[ib].at[j]], msgs[mb].at[j],
                                 gsem)

        def gwait(mb):
            for j in range(KRA):
                pltpu.make_async_copy(table.at[sidx[0].at[j]],
                                      msgs[mb].at[j], gsem).wait()

        def scat(ib, mb):
            for j in range(KRA):
                pltpu.async_copy(msgs[mb].at[j], acc.at[didx[ib].at[j]],
                                 ssem, add=True)

        def swait():
            for j in range(KRA):
                pltpu.make_async_copy(msgs[0].at[j],
                                      acc.at[didx[0].at[j]], ssem).wait()

        iload(0, 0)
        iwait(0)
        gathers(0, 0)
        iload(1, 1)
        iwait(1)
        iload(2, 2)
        gwait(0)
        gathers(1, 1)
        scat(0, 0)
        swait()
        iwait(2)
        iload(3, 0)
        gwait(1)
        gathers(2, 0)
        scat(1, 1)

        def body_m(m, _):
            t0 = 2 + 6 * m
            for i in range(6):
                mb = i % 2
                ib = (i + 2) % 3
                swait()
                iwait(i % 3)
                iload(t0 + i + 2, (i + 1) % 3)
                gwait(mb)
                gathers(i % 3, (i + 1) % 2)
                scat(ib, mb)
            return 0

        lax.fori_loop(0, (PHASES - 4) // 6, body_m, 0)

        swait()
        iwait(0)
        gwait(0)
        gathers(0, 1)
        scat(2, 0)
        swait()
        gwait(1)
        scat(0, 1)
        swait()

        plsc.subcore_barrier()
        pltpu.sync_copy(acc.at[pl.ds(s * ZROWS, ZROWS), :],
                        aggp.at[c, pl.ds(s * ZROWS, ZROWS), :])

    scratch = [
        pltpu.VMEM_SHARED((NPAD, W), jnp.float32),
        pltpu.VMEM((KRA, 128), jnp.int32),
        pltpu.VMEM((KRA, 128), jnp.int32),
        pltpu.VMEM((KRA, 128), jnp.int32),
        pltpu.VMEM((KRA, 128), jnp.int32),
        pltpu.VMEM((KRA, 128), jnp.int32),
        pltpu.VMEM((KRA, 128), jnp.int32),
        pltpu.VMEM((KRA, 128, W), jnp.float32),
        pltpu.VMEM((KRA, 128, W), jnp.float32),
        pltpu.SemaphoreType.DMA,
        pltpu.SemaphoreType.DMA,
        pltpu.SemaphoreType.DMA,
    ]

    return pl.kernel(
        body,
        out_type=jax.ShapeDtypeStruct((NC, NPAD, W), jnp.float32),
        mesh=_MESH,
        compiler_params=_SC_PARAMS,
        scratch_types=scratch,
    )


_agg16 = _make_agg(HID)


def _bs(shape, idx):
    return pl.BlockSpec(shape, idx)


_row = lambda i: (i, 0)
_rep = lambda i: (0, 0)
_r3 = lambda i: (0, i, 0)


def _tca_body(x, cW1p, cb1, cW2, cb2, gW1, gb1, gv, gb, ctxpack_o):
    xv = x[...]
    ch = jnp.maximum(jnp.dot(xv, cW1p[...], precision=_HIGH) + cb1[...], 0.0)
    ctx = jnp.dot(ch, cW2[...], precision=_HIGH) + cb2[...]
    gh = jnp.maximum(jnp.dot(xv, gW1[...], precision=_HIGH) + gb1[...], 0.0)
    dl = jnp.dot(gh, gv[...], precision=_HIGH) + gb[...]
    w0 = 1.0 / (1.0 + jnp.exp(-dl))
    pre = (1.0 - w0) * ctx
    ctxpack_o[...] = jnp.concatenate(
        [pre, jnp.broadcast_to(w0, (BLK, OUT))], axis=1)


_tca = pl.pallas_call(
    _tca_body,
    grid=(GRID,),
    in_specs=[
        _bs((BLK, FULL_DIM), _row),
        _bs((FULL_DIM, HID), _rep), _bs((1, HID), _rep),
        _bs((HID, OUT), _rep), _bs((1, OUT), _rep),
        _bs((FULL_DIM, HID), _rep), _bs((1, HID), _rep),
        _bs((HID, 1), _rep), _bs((1, 1), _rep),
    ],
    out_specs=[_bs((BLK, FULL_DIM), _row)],
    out_shape=[jax.ShapeDtypeStruct((NPAD, FULL_DIM), jnp.float32)],
)


def _tcb0_body(degp, dinv_o):
    dinv_o[...] = lax.rsqrt(degp[0] + degp[1] + 1.0)


_tcb0 = pl.pallas_call(
    _tcb0_body,
    grid=(1,),
    in_specs=[_bs((NC, LROWS, 128), _r3)],
    out_specs=[_bs((LROWS, 128), _row)],
    out_shape=[jax.ShapeDtypeStruct((LROWS, 128), jnp.float32)],
)


def _tcb_body(x16p, dsp, t1_o):
    t1_o[...] = x16p[...] * dsp[...]


_tcb = pl.pallas_call(
    _tcb_body,
    grid=(GRID,),
    in_specs=[_bs((PACK, 128), _row), _bs((PACK, 128), _row)],
    out_specs=[_bs((PACK, 128), _row)],
    out_shape=[jax.ShapeDtypeStruct((LIN, 128), jnp.float32)],
)


def _tcc_body(gp, t1, dsp, W1B, b1t, t2_o):
    dv = dsp[...]
    s1 = (gp[0] + gp[1] + t1[...]) * dv
    h = jnp.maximum(jnp.dot(s1, W1B[...], precision=_HIGH) + b1t[...], 0.0)
    t2_o[...] = h * dv


_tcc = pl.pallas_call(
    _tcc_body,
    grid=(GRID,),
    in_specs=[
        _bs((NC, PACK, 128), _r3),
        _bs((PACK, 128), _row),
        _bs((PACK, 128), _row),
        _bs((128, 128), _rep), _bs((1, 128), _rep),
    ],
    out_specs=[_bs((PACK, 128), _row)],
    out_shape=[jax.ShapeDtypeStruct((LIN, 128), jnp.float32)],
)


def _tcd_body(gp, t2, dsp, ctxpack, W2, b2, out_o):
    s2p = (gp[0] + gp[1] + t2[...]) * dsp[...]
    rows = lax.broadcasted_iota(jnp.int32, (BLK, 128), 0)
    cols = lax.broadcasted_iota(jnp.int32, (BLK, 128), 1)
    P = (cols == rows // 8).astype(jnp.float32)
    M = ((cols // HID) == (rows % 8)).astype(jnp.float32)
    gr = lax.broadcasted_iota(jnp.int32, (128, HID), 0)
    gc = lax.broadcasted_iota(jnp.int32, (128, HID), 1)
    G = ((gr % HID) == gc).astype(jnp.float32)
    s2 = jnp.dot(jnp.dot(P, s2p, precision=_HIGH) * M, G, precision=_HIGH)
    go = jnp.dot(s2, W2[...], precision=_HIGH) + b2[...]
    cp = ctxpack[...]
    w0 = cp[:, OUT:OUT + 1]
    pre = cp[:, :OUT]
    out_o[...] = w0 * go + pre


_tcd = pl.pallas_call(
    _tcd_body,
    grid=(GRID,),
    in_specs=[
        _bs((NC, PACK, 128), _r3),
        _bs((PACK, 128), _row),
        _bs((PACK, 128), _row),
        _bs((BLK, FULL_DIM), _row),
        _bs((HID, OUT), _rep), _bs((1, OUT), _rep),
    ],
    out_specs=[_bs((BLK, OUT), _row)],
    out_shape=[jax.ShapeDtypeStruct((N, OUT), jnp.float32)],
)


def kernel(x, edge_index, W1, b1, W2, b2, cW1, cb1, cW2, cb2, gW1, gb1,
           gW2, gb2):
    pad = jnp.full((EPAD - E,), SENT, jnp.int32)
    src2 = jnp.concatenate([edge_index[0], pad]).reshape(EPAD // 128, 128)
    dst2 = jnp.concatenate([edge_index[1], pad]).reshape(EPAD // 128, 128)

    cW1p = jnp.zeros((FULL_DIM, HID), jnp.float32).at[GRAPH_DIM:, :].set(cW1)
    W1p = jnp.zeros((HID, HID), jnp.float32).at[:GRAPH_DIM, :].set(W1)
    W1B = jnp.kron(jnp.eye(8, dtype=jnp.float32), W1p)
    b1t = jnp.tile(b1.reshape(1, HID), (1, 8))
    gv = (gW2[:, 0] - gW2[:, 1]).reshape(HID, 1)
    gb = (gb2[0] - gb2[1]).reshape(1, 1)

    xp = jnp.pad(x, ((0, NPAD - N), (0, 0)))
    x16p = jnp.pad(x[:, :HID], ((0, NPAD - N), (0, 0))).reshape(LIN, 128)

    zrow1 = jnp.zeros((ZROWS,), jnp.float32)
    zrow16 = jnp.zeros((ZROWS, HID), jnp.float32)
    degp = _deg_kernel(dst2, zrow1)
    ctxpack = _tca(xp, cW1p, cb1.reshape(1, HID), cW2, cb2.reshape(1, OUT),
                   gW1, gb1.reshape(1, HID), gv, gb)[0]

    dinvL = _tcb0(degp.reshape(NC, LROWS, 128))[0]
    dsp = jnp.broadcast_to(dinvL.reshape(NPAD, 1), (NPAD, HID)).reshape(
        LIN, 128)

    t1 = _tcb(x16p, dsp)[0]

    g1 = _agg16(t1.reshape(NPAD, HID), src2, dst2, zrow16)

    t2 = _tcc(g1.reshape(NC, LIN, 128), t1, dsp, W1B, b1t)[0]

    g2 = _agg16(t2.reshape(NPAD, HID), src2, dst2, zrow16)

    out = _tcd(g2.reshape(NC, LIN, 128), t2, dsp, ctxpack, W2,
               b2.reshape(1, OUT))[0]
    return out


# speedup vs baseline: 70.2506x; 1.0909x over previous
import jax
import jax.numpy as jnp
from jax import lax
from jax.experimental import pallas as pl
from jax.experimental.pallas import tpu as pltpu
from jax.experimental.pallas import tpu_sc as plsc

N = 100000
E = 3200000
FULL_DIM = 128
GRAPH_DIM = 3
HID = 16
OUT = 64

NC = 2
NS = 16
NW = NC * NS

NPAD = 100352
EPAD = NW * NPAD
TPW = EPAD // NW
RPW = TPW // 128
ZROWS = NPAD // NS
SENT = NPAD - 1

BLK = 1024
GRID = NPAD // BLK
PACK = BLK * HID // 128
LIN = NPAD * HID // 128
LROWS = NPAD // 128

_MESH = plsc.VectorSubcoreMesh(core_axis_name="c", subcore_axis_name="s")
_HIGH = jax.lax.Precision.HIGHEST
_SC_PARAMS = pltpu.CompilerParams(use_tc_tiling_on_sc=False)


def _fill(ref, nvec, value):
    v = jnp.full((16,), value, jnp.float32)

    def body(i, _):
        ref[pl.ds(i * 16, 16)] = v
        return 0

    lax.fori_loop(0, nvec, body, 0)


KROWS = 8
ITERS = RPW // KROWS


def _deg_body(dst2, zrow, degp, didx, ones_v, acc):
    c = lax.axis_index("c")
    s = lax.axis_index("s")
    wid = c * NS + s

    _fill(ones_v, 8, 1.0)
    pltpu.sync_copy(zrow, acc.at[pl.ds(s * ZROWS, ZROWS)])
    plsc.subcore_barrier()

    def body(it, _):
        row0 = wid * RPW + it * KROWS
        pltpu.sync_copy(dst2.at[pl.ds(row0, KROWS)], didx)
        for j in range(KROWS):
            pltpu.sync_copy(ones_v, acc.at[didx.at[j]], add=True)
        return 0

    lax.fori_loop(0, ITERS, body, 0)
    plsc.subcore_barrier()
    pltpu.sync_copy(acc.at[pl.ds(s * ZROWS, ZROWS)],
                    degp.at[c, pl.ds(s * ZROWS, ZROWS)])


_deg_kernel = pl.kernel(
    _deg_body,
    out_type=jax.ShapeDtypeStruct((NC, NPAD), jnp.float32),
    mesh=_MESH,
    compiler_params=_SC_PARAMS,
    scratch_types=[
        pltpu.VMEM((KROWS, 128), jnp.int32),
        pltpu.VMEM((128,), jnp.float32),
        pltpu.VMEM_SHARED((NPAD,), jnp.float32),
    ],
)


KRA = 4
PHASES = RPW // KRA


def _make_agg(W):
    def body(table, src2, dst2, zrows, aggp, acc,
             si0, si1, si2, di0, di1, di2, ms0, ms1,
             isem, gsem, ssem):
        sidx = [si0, si1, si2]
        didx = [di0, di1, di2]
        msgs = [ms0, ms1]
        c = lax.axis_index("c")
        s = lax.axis_index("s")
        wid = c * NS + s
        base = wid * RPW

        pltpu.sync_copy(zrows, acc.at[pl.ds(s * ZROWS, ZROWS), :])
        plsc.subcore_barrier()

        def iload(t, ib):
            row0 = base + t * KRA
            pltpu.async_copy(src2.at[pl.ds(row0, KRA)], sidx[ib], isem)
            pltpu.async_copy(dst2.at[pl.ds(row0, KRA)], didx[ib], isem)

        def iwait(ib):
            pltpu.make_async_copy(src2.at[pl.ds(0, KRA)], sidx[ib],
                                  isem).wait()
            pltpu.make_async_copy(dst2.at[pl.ds(0, KRA)], didx[ib],
                                  isem).wait()

        def gathers(ib, mb):
            for j in range(KRA):
                pltpu.async_copy(table.at[sidx[ib].at[j]], msgs[mb].at[j],
                                 gsem)

        def gwait(mb):
            for j in range(KRA):
                pltpu.make_async_copy(table.at[sidx[0].at[j]],
                                      msgs[mb].at[j], gsem).wait()

        def scat(ib, mb):
            for j in range(KRA):
                pltpu.async_copy(msgs[mb].at[j], acc.at[didx[ib].at[j]],
                                 ssem, add=True)

        def swait():
            for j in range(KRA):
                pltpu.make_async_copy(msgs[0].at[j],
                                      acc.at[didx[0].at[j]], ssem).wait()

        iload(0, 0)
        iwait(0)
        gathers(0, 0)
        iload(1, 1)
        iwait(1)
        iload(2, 2)
        gwait(0)
        gathers(1, 1)
        scat(0, 0)
        swait()
        iwait(2)
        iload(3, 0)
        gwait(1)
        gathers(2, 0)
        scat(1, 1)

        def body_m(m, _):
            t0 = 2 + 6 * m
            for i in range(6):
                mb = i % 2
                ib = (i + 2) % 3
                swait()
                iwait(i % 3)
                iload(t0 + i + 2, (i + 1) % 3)
                gwait(mb)
                gathers(i % 3, (i + 1) % 2)
                scat(ib, mb)
            return 0

        lax.fori_loop(0, (PHASES - 4) // 6, body_m, 0)

        swait()
        iwait(0)
        gwait(0)
        gathers(0, 1)
        scat(2, 0)
        swait()
        gwait(1)
        scat(0, 1)
        swait()

        plsc.subcore_barrier()
        pltpu.sync_copy(acc.at[pl.ds(s * ZROWS, ZROWS), :],
                        aggp.at[c, pl.ds(s * ZROWS, ZROWS), :])

    scratch = [
        pltpu.VMEM_SHARED((NPAD, W), jnp.float32),
        pltpu.VMEM((KRA, 128), jnp.int32),
        pltpu.VMEM((KRA, 128), jnp.int32),
        pltpu.VMEM((KRA, 128), jnp.int32),
        pltpu.VMEM((KRA, 128), jnp.int32),
        pltpu.VMEM((KRA, 128), jnp.int32),
        pltpu.VMEM((KRA, 128), jnp.int32),
        pltpu.VMEM((KRA, 128, W), jnp.float32),
        pltpu.VMEM((KRA, 128, W), jnp.float32),
        pltpu.SemaphoreType.DMA,
        pltpu.SemaphoreType.DMA,
        pltpu.SemaphoreType.DMA,
    ]

    return pl.kernel(
        body,
        out_type=jax.ShapeDtypeStruct((NC, NPAD, W), jnp.float32),
        mesh=_MESH,
        compiler_params=_SC_PARAMS,
        scratch_types=scratch,
    )


_agg16 = _make_agg(HID)


def _bs(shape, idx):
    return pl.BlockSpec(shape, idx)


_row = lambda i: (i, 0)
_rep = lambda i: (0, 0)
_r3 = lambda i: (0, i, 0)


def _tca_body(x, cW1p, cb1, cW2, cb2, gW1, gb1, gv, gb, ctxpack_o):
    xv = x[...]
    ch = jnp.maximum(jnp.dot(xv, cW1p[...], precision=_HIGH) + cb1[...], 0.0)
    ctx = jnp.dot(ch, cW2[...], precision=_HIGH) + cb2[...]
    gh = jnp.maximum(jnp.dot(xv, gW1[...], precision=_HIGH) + gb1[...], 0.0)
    dl = jnp.dot(gh, gv[...], precision=_HIGH) + gb[...]
    w0 = 1.0 / (1.0 + jnp.exp(-dl))
    pre = (1.0 - w0) * ctx
    ctxpack_o[...] = jnp.concatenate(
        [pre, jnp.broadcast_to(w0, (BLK, OUT))], axis=1)


_tca = pl.pallas_call(
    _tca_body,
    grid=(GRID,),
    in_specs=[
        _bs((BLK, FULL_DIM), _row),
        _bs((FULL_DIM, HID), _rep), _bs((1, HID), _rep),
        _bs((HID, OUT), _rep), _bs((1, OUT), _rep),
        _bs((FULL_DIM, HID), _rep), _bs((1, HID), _rep),
        _bs((HID, 1), _rep), _bs((1, 1), _rep),
    ],
    out_specs=[_bs((BLK, FULL_DIM), _row)],
    out_shape=[jax.ShapeDtypeStruct((NPAD, FULL_DIM), jnp.float32)],
)


def _tcb0_body(degp, dinv_o):
    dinv_o[...] = lax.rsqrt(degp[0] + degp[1] + 1.0)


_tcb0 = pl.pallas_call(
    _tcb0_body,
    grid=(1,),
    in_specs=[_bs((NC, LROWS, 128), _r3)],
    out_specs=[_bs((LROWS, 128), _row)],
    out_shape=[jax.ShapeDtypeStruct((LROWS, 128), jnp.float32)],
)


def _tcb_body(x16p, dsp, t1_o):
    t1_o[...] = x16p[...] * dsp[...]


RB = LIN // 14

_tcb = pl.pallas_call(
    _tcb_body,
    grid=(14,),
    in_specs=[_bs((RB, 128), _row), _bs((RB, 128), _row)],
    out_specs=[_bs((RB, 128), _row)],
    out_shape=[jax.ShapeDtypeStruct((LIN, 128), jnp.float32)],
)


def _tcc_body(gp, t1, dsp, W1B, b1t, t2_o):
    dv = dsp[...]
    s1 = (gp[0] + gp[1] + t1[...]) * dv
    h = jnp.maximum(jnp.dot(s1, W1B[...], precision=_HIGH) + b1t[...], 0.0)
    t2_o[...] = h * dv


_tcc = pl.pallas_call(
    _tcc_body,
    grid=(14,),
    in_specs=[
        _bs((NC, RB, 128), _r3),
        _bs((RB, 128), _row),
        _bs((RB, 128), _row),
        _bs((128, 128), _rep), _bs((1, 128), _rep),
    ],
    out_specs=[_bs((RB, 128), _row)],
    out_shape=[jax.ShapeDtypeStruct((LIN, 128), jnp.float32)],
)


def _tcd_body(gp, t2, dsp, ctxpack, W2t, b2, out_o):
    s2p = (gp[0] + gp[1] + t2[...]) * dsp[...]
    rows = lax.broadcasted_iota(jnp.int32, (BLK, 128), 0)
    cols = lax.broadcasted_iota(jnp.int32, (BLK, 128), 1)
    P = (cols == rows // 8).astype(jnp.float32)
    M = ((cols // HID) == (rows % 8)).astype(jnp.float32)
    go = jnp.dot(jnp.dot(P, s2p, precision=_HIGH) * M, W2t[...],
                 precision=_HIGH) + b2[...]
    cp = ctxpack[...]
    w0 = cp[:, OUT:OUT + 1]
    pre = cp[:, :OUT]
    out_o[...] = w0 * go + pre


_tcd = pl.pallas_call(
    _tcd_body,
    grid=(GRID,),
    in_specs=[
        _bs((NC, PACK, 128), _r3),
        _bs((PACK, 128), _row),
        _bs((PACK, 128), _row),
        _bs((BLK, FULL_DIM), _row),
        _bs((128, OUT), _rep), _bs((1, OUT), _rep),
    ],
    out_specs=[_bs((BLK, OUT), _row)],
    out_shape=[jax.ShapeDtypeStruct((N, OUT), jnp.float32)],
)


def kernel(x, edge_index, W1, b1, W2, b2, cW1, cb1, cW2, cb2, gW1, gb1,
           gW2, gb2):
    pad = jnp.full((EPAD - E,), SENT, jnp.int32)
    src2 = jnp.concatenate([edge_index[0], pad]).reshape(EPAD // 128, 128)
    dst2 = jnp.concatenate([edge_index[1], pad]).reshape(EPAD // 128, 128)

    cW1p = jnp.zeros((FULL_DIM, HID), jnp.float32).at[GRAPH_DIM:, :].set(cW1)
    W1p = jnp.zeros((HID, HID), jnp.float32).at[:GRAPH_DIM, :].set(W1)
    W1B = jnp.kron(jnp.eye(8, dtype=jnp.float32), W1p)
    b1t = jnp.tile(b1.reshape(1, HID), (1, 8))
    gv = (gW2[:, 0] - gW2[:, 1]).reshape(HID, 1)
    gb = (gb2[0] - gb2[1]).reshape(1, 1)

    xp = jnp.pad(x, ((0, NPAD - N), (0, 0)))
    x16p = jnp.pad(x[:, :HID], ((0, NPAD - N), (0, 0))).reshape(LIN, 128)

    zrow1 = jnp.zeros((ZROWS,), jnp.float32)
    zrow16 = jnp.zeros((ZROWS, HID), jnp.float32)
    degp = _deg_kernel(dst2, zrow1)
    ctxpack = _tca(xp, cW1p, cb1.reshape(1, HID), cW2, cb2.reshape(1, OUT),
                   gW1, gb1.reshape(1, HID), gv, gb)[0]

    dinvL = _tcb0(degp.reshape(NC, LROWS, 128))[0]
    dsp = jnp.broadcast_to(dinvL.reshape(NPAD, 1), (NPAD, HID)).reshape(
        LIN, 128)

    t1 = _tcb(x16p, dsp)[0]

    g1 = _agg16(t1.reshape(NPAD, HID), src2, dst2, zrow16)

    t2 = _tcc(g1.reshape(NC, LIN, 128), t1, dsp, W1B, b1t)[0]

    g2 = _agg16(t2.reshape(NPAD, HID), src2, dst2, zrow16)

    out = _tcd(g2.reshape(NC, LIN, 128), t2, dsp, ctxpack,
               jnp.tile(W2, (8, 1)), b2.reshape(1, OUT))[0]
    return out
